# Initial kernel scaffold; baseline (speedup 1.0000x reference)
import jax
import jax.numpy as jnp
from jax.experimental import pallas as pl


def kernel(x, edge_index, edge_attr, g_true_0, g_true_1, W_lin, b_lin, w_e, b_e, W1, b1, Wc1, W2, b2, Wc2):
    raise NotImplementedError("write your pallas kernel here")


# speedup vs baseline: 3.6137x; 3.6137x over previous
import functools

import jax
import jax.numpy as jnp
from jax import lax
from jax.experimental import pallas as pl
from jax.experimental.pallas import tpu as pltpu
from jax.experimental.pallas import tpu_sc as plsc

N_NODES = 10000
N_EDGES = 320000
D = 128
K0 = 100
K1 = 10

NC = 2
NS = 16
NW = NC * NS
EPW = N_EDGES // NW
CH = 80
NCHUNK = EPW // CH
ROWS_PER_TILE = N_NODES // NS


def _seg_sum_body(table_hbm, gidx_hbm, sidx_hbm, gate_hbm, out_hbm,
                  gidx_v, sidx_v, gate_v, rows_v, acc_sh, sem):
    c = lax.axis_index("c")
    s = lax.axis_index("s")
    wid = c * NS + s

    zero16 = jnp.zeros((16,), jnp.float32)

    @pl.loop(0, CH)
    def _zero_rows(r):
        for j in range(D // 16):
            rows_v[r, pl.ds(j * 16, 16)] = zero16

    n_row_chunks = N_NODES // CH

    @pl.loop(0, (n_row_chunks + NS - 1) // NS)
    def _zero_acc(k):
        rc = s + k * NS

        @pl.when(rc < n_row_chunks)
        def _():
            pltpu.sync_copy(rows_v, acc_sh.at[pl.ds(rc * CH, CH)])

    plsc.subcore_barrier()

    @pl.loop(0, NCHUNK)
    def _chunk(ci):
        base = wid * EPW + ci * CH
        pltpu.sync_copy(gidx_hbm.at[pl.ds(base, CH)], gidx_v)
        pltpu.sync_copy(gate_hbm.at[pl.ds(base, CH)], gate_v)
        pltpu.async_copy(table_hbm.at[gidx_v], rows_v, sem).wait()

        @pl.loop(0, CH // 16)
        def _scale(rb):
            g16 = gate_v[pl.ds(rb * 16, 16)]
            for j in range(16):
                g = g16[j]
                r = rb * 16 + j
                for k in range(D // 16):
                    rows_v[r, pl.ds(k * 16, 16)] = rows_v[r, pl.ds(k * 16, 16)] * g

        pltpu.sync_copy(sidx_hbm.at[pl.ds(base, CH)], sidx_v)
        pltpu.sync_copy(rows_v, acc_sh.at[sidx_v], add=True)

    plsc.subcore_barrier()

    @pl.loop(0, (n_row_chunks + NS - 1) // NS)
    def _dump(k):
        rc = s + k * NS

        @pl.when(rc < n_row_chunks)
        def _():
            pltpu.sync_copy(acc_sh.at[pl.ds(rc * CH, CH)],
                            out_hbm.at[c, pl.ds(rc * CH, CH)])


@functools.lru_cache(maxsize=1)
def _make_seg_sum_sc():
    return pl.kernel(
        _seg_sum_body,
        out_type=jax.ShapeDtypeStruct((NC, N_NODES, D), jnp.float32),
        mesh=plsc.VectorSubcoreMesh(core_axis_name="c", subcore_axis_name="s"),
        scratch_types=[
            pltpu.VMEM((CH,), jnp.int32),
            pltpu.VMEM((CH,), jnp.int32),
            pltpu.VMEM((CH,), jnp.float32),
            pltpu.VMEM((CH, D), jnp.float32),
            pltpu.VMEM_SHARED((N_NODES, D), jnp.float32),
            pltpu.SemaphoreType.DMA,
        ],
    )


def _segment_sum_sc(table, gather_idx, scatter_idx, gate):
    return _make_seg_sum_sc()(table, gather_idx, scatter_idx, gate)


ROWB = 400


def _embed_body(x_ref, w_ref, b_ref, out_ref):
    out_ref[...] = jnp.dot(x_ref[...], w_ref[...],
                           preferred_element_type=jnp.float32) + b_ref[...]


def _embed(x, W_lin, b_lin):
    grid = (N_NODES // ROWB,)
    return pl.pallas_call(
        _embed_body,
        grid=grid,
        in_specs=[
            pl.BlockSpec((ROWB, D), lambda i: (i, 0)),
            pl.BlockSpec((D, D), lambda i: (0, 0)),
            pl.BlockSpec((1, D), lambda i: (0, 0)),
        ],
        out_specs=pl.BlockSpec((ROWB, D), lambda i: (i, 0)),
        out_shape=jax.ShapeDtypeStruct((N_NODES, D), jnp.float32),
    )(x, W_lin, b_lin.reshape(1, D))


EB = 32000


def _gate_body(c0, c1, c2, c3, w_ref, out_ref):
    z = (c0[...] * w_ref[0, 0] + c1[...] * w_ref[0, 1]
         + c2[...] * w_ref[0, 2] + c3[...] * w_ref[0, 3] + w_ref[0, 4])
    out_ref[...] = 1.0 / (1.0 + jnp.exp(-z))


def _gate(edge_attr, w_e, b_e):
    rows = N_EDGES // 128
    cols = [edge_attr[:, k].reshape(rows, 128) for k in range(4)]
    wb = jnp.concatenate([w_e.reshape(-1), b_e.reshape(-1)]).reshape(1, 5)
    g2 = pl.pallas_call(
        _gate_body,
        in_specs=[pl.BlockSpec((rows, 128), lambda: (0, 0))] * 4
        + [pl.BlockSpec(memory_space=pltpu.MemorySpace.SMEM)],
        out_specs=pl.BlockSpec((rows, 128), lambda: (0, 0)),
        out_shape=jax.ShapeDtypeStruct((rows, 128), jnp.float32),
    )(*cols, wb)
    return g2.reshape(N_EDGES)


def _mid_body(x0_ref, m_ref, w1a_ref, w1b_ref, b1_ref, wc1_ref,
              g0_ref, h1_ref, s1_ref, x1_ref, l0_ref):
    i = pl.program_id(0)
    msg = m_ref[0] + m_ref[1]
    h1 = jnp.dot(x0_ref[...], w1a_ref[...], preferred_element_type=jnp.float32)
    h1 += jnp.dot(msg, w1b_ref[...], preferred_element_type=jnp.float32)
    h1 = jnp.maximum(h1 + b1_ref[...], 0.0)
    h1_ref[...] = h1
    logits = jnp.dot(h1, wc1_ref[...], preferred_element_type=jnp.float32)
    col = lax.broadcasted_iota(jnp.int32, logits.shape, 1)
    logits = jnp.where(col < K0, logits, -jnp.inf)
    m = jnp.max(logits, axis=-1, keepdims=True)
    e = jnp.exp(logits - m)
    s1 = e / jnp.sum(e, axis=-1, keepdims=True)
    s1_ref[...] = s1
    d = s1 - g0_ref[...]
    part = jnp.sum(d * d)
    x1_blk = jnp.dot(s1.T, h1, preferred_element_type=jnp.float32)

    @pl.when(i == 0)
    def _():
        x1_ref[...] = x1_blk
        l0_ref[0, 0] = part

    @pl.when(i != 0)
    def _():
        x1_ref[...] += x1_blk
        l0_ref[0, 0] += part


def _mid(x0, msg2, W1, b1, Wc1p, g0p):
    grid = (N_NODES // ROWB,)
    return pl.pallas_call(
        _mid_body,
        grid=grid,
        in_specs=[
            pl.BlockSpec((ROWB, D), lambda i: (i, 0)),
            pl.BlockSpec((NC, ROWB, D), lambda i: (0, i, 0)),
            pl.BlockSpec((D, D), lambda i: (0, 0)),
            pl.BlockSpec((D, D), lambda i: (0, 0)),
            pl.BlockSpec((1, D), lambda i: (0, 0)),
            pl.BlockSpec((D, D), lambda i: (0, 0)),
            pl.BlockSpec((ROWB, D), lambda i: (i, 0)),
        ],
        out_specs=[
            pl.BlockSpec((ROWB, D), lambda i: (i, 0)),
            pl.BlockSpec((ROWB, D), lambda i: (i, 0)),
            pl.BlockSpec((D, D), lambda i: (0, 0)),
            pl.BlockSpec(memory_space=pltpu.MemorySpace.SMEM),
        ],
        out_shape=[
            jax.ShapeDtypeStruct((N_NODES, D), jnp.float32),
            jax.ShapeDtypeStruct((N_NODES, D), jnp.float32),
            jax.ShapeDtypeStruct((D, D), jnp.float32),
            jax.ShapeDtypeStruct((1, 1), jnp.float32),
        ],
    )(x0, msg2, W1[:D], W1[D:], b1.reshape(1, D), Wc1p, g0p)


def _fin_body(s1_ref, b_ref, x1_ref, w2a_ref, w2b_ref, b2_ref,
              wc2_ref, g1_ref, a1_ref, x2_ref, l1_ref, acc):
    i = pl.program_id(0)
    nblk = pl.num_programs(0)
    bsum = b_ref[0] + b_ref[1]
    blk = jnp.dot(s1_ref[...].T, bsum, preferred_element_type=jnp.float32)

    @pl.when(i == 0)
    def _():
        acc[...] = blk

    @pl.when(i != 0)
    def _():
        acc[...] += blk

    @pl.when(i == nblk - 1)
    def _():
        a1 = acc[...]
        a1_ref[...] = a1
        x1 = x1_ref[...]
        msg2 = jnp.dot(a1, x1, preferred_element_type=jnp.float32)
        h2 = jnp.dot(x1, w2a_ref[...], preferred_element_type=jnp.float32)
        h2 += jnp.dot(msg2, w2b_ref[...], preferred_element_type=jnp.float32)
        h2 = jnp.maximum(h2 + b2_ref[...], 0.0)
        row = lax.broadcasted_iota(jnp.int32, h2.shape, 0)
        h2 = jnp.where(row < K0, h2, 0.0)
        logits = jnp.dot(h2, wc2_ref[...], preferred_element_type=jnp.float32)
        col = lax.broadcasted_iota(jnp.int32, logits.shape, 1)
        valid = (col < K1) & (row < K0)
        logits = jnp.where(col < K1, logits, -jnp.inf)
        m = jnp.max(logits, axis=-1, keepdims=True)
        e = jnp.exp(logits - m)
        s2 = e / jnp.sum(e, axis=-1, keepdims=True)
        s2 = jnp.where(row < K0, s2, 0.0)
        d = jnp.where(valid, s2 - g1_ref[...], 0.0)
        l1_ref[0, 0] = jnp.sum(d * d) / (K0 * K1)
        x2_ref[...] = jnp.dot(s2.T, h2, preferred_element_type=jnp.float32)


def _final(s1p, bparts, x1, W2, b2, Wc2p, g1p):
    grid = (N_NODES // ROWB,)
    return pl.pallas_call(
        _fin_body,
        grid=grid,
        in_specs=[
            pl.BlockSpec((ROWB, D), lambda i: (i, 0)),
            pl.BlockSpec((NC, ROWB, D), lambda i: (0, i, 0)),
            pl.BlockSpec((D, D), lambda i: (0, 0)),
            pl.BlockSpec((D, D), lambda i: (0, 0)),
            pl.BlockSpec((D, D), lambda i: (0, 0)),
            pl.BlockSpec((1, D), lambda i: (0, 0)),
            pl.BlockSpec((D, D), lambda i: (0, 0)),
            pl.BlockSpec((D, D), lambda i: (0, 0)),
        ],
        out_specs=[
            pl.BlockSpec((D, D), lambda i: (0, 0)),
            pl.BlockSpec((D, D), lambda i: (0, 0)),
            pl.BlockSpec(memory_space=pltpu.MemorySpace.SMEM),
        ],
        out_shape=[
            jax.ShapeDtypeStruct((D, D), jnp.float32),
            jax.ShapeDtypeStruct((D, D), jnp.float32),
            jax.ShapeDtypeStruct((1, 1), jnp.float32),
        ],
        scratch_shapes=[pltpu.VMEM((D, D), jnp.float32)],
    )(s1p, bparts, x1, W2[:D], W2[D:], b2.reshape(1, D), Wc2p, g1p)


def kernel(x, edge_index, edge_attr, g_true_0, g_true_1, W_lin, b_lin,
           w_e, b_e, W1, b1, Wc1, W2, b2, Wc2):
    src = edge_index[0].astype(jnp.int32)
    dst = edge_index[1].astype(jnp.int32)

    x0 = _embed(x, W_lin, b_lin)
    gate = _gate(edge_attr, w_e, b_e)

    msg2 = _segment_sum_sc(x0, src, dst, gate)

    Wc1p = jnp.pad(Wc1, ((0, 0), (0, D - K0)))
    g0p = jnp.pad(g_true_0, ((0, 0), (0, D - K0)))
    h1, s1p, x1, l0 = _mid(x0, msg2, W1, b1, Wc1p, g0p)
    loss0 = l0[0, 0] / (N_NODES * K0)

    bparts = _segment_sum_sc(s1p, dst, src, gate)

    Wc2p = jnp.pad(Wc2, ((0, 0), (0, D - K1)))
    g1p = jnp.pad(g_true_1, ((0, D - K0), (0, D - K1)))
    a1, x2p, l1 = _final(s1p, bparts, x1, W2, b2, Wc2p, g1p)

    x2 = x2p[:K1, :]
    return jnp.concatenate([x2.reshape(-1), jnp.stack([loss0, l1[0, 0]])])


# speedup vs baseline: 7.1994x; 1.9922x over previous
import functools

import jax
import jax.numpy as jnp
from jax import lax
from jax.experimental import pallas as pl
from jax.experimental.pallas import tpu as pltpu
from jax.experimental.pallas import tpu_sc as plsc

N_NODES = 10000
N_EDGES = 320000
D = 128
K0 = 100
K1 = 10

NC = 2
NS = 16
NW = NC * NS
EPW = N_EDGES // NW
CH = 64
NFULL = EPW // CH
REM = EPW - NFULL * CH
ZCH = 80


def _seg_sum_body(table_hbm, gidx_hbm, sidx_hbm, gate_hbm, out_hbm,
                  gidx_all, sidx_all, gate_all, rows0, rows1, sidx_small,
                  sidx_rem, acc_sh, semg0, semg1):
    c = lax.axis_index("c")
    s = lax.axis_index("s")
    wid = c * NS + s
    ebase = wid * EPW

    pltpu.sync_copy(gidx_hbm.at[pl.ds(ebase, EPW)], gidx_all)
    pltpu.sync_copy(sidx_hbm.at[pl.ds(ebase, EPW)], sidx_all)
    pltpu.sync_copy(gate_hbm.at[pl.ds(ebase, EPW)], gate_all)

    zero16 = jnp.zeros((16,), jnp.float32)

    @pl.loop(0, ZCH)
    def _zero_rows(r):
        for j in range(D // 16):
            rows0[r, pl.ds(j * 16, 16)] = zero16

    n_row_chunks = N_NODES // ZCH

    @pl.loop(0, (n_row_chunks + NS - 1) // NS)
    def _zero_acc(k):
        rc = s + k * NS

        @pl.when(rc < n_row_chunks)
        def _():
            pltpu.sync_copy(rows0.at[pl.ds(0, ZCH)], acc_sh.at[pl.ds(rc * ZCH, ZCH)])

    plsc.subcore_barrier()

    rows = (rows0, rows1)
    semg = (semg0, semg1)

    def start_gather(cc, b):
        pltpu.async_copy(table_hbm.at[gidx_all.at[pl.ds(cc * CH, CH)]],
                         rows[b], semg[b])

    def wait_gather(b):
        pltpu.make_async_copy(table_hbm.at[gidx_all.at[pl.ds(0, CH)]],
                              rows[b], semg[b]).wait()

    def process(cc, b):
        rv = rows[b]

        @pl.loop(0, CH // 16)
        def _scale(rb):
            g16 = gate_all[pl.ds(cc * CH + rb * 16, 16)]
            for j in range(16):
                g = g16[j]
                r = rb * 16 + j
                for k2 in range(D // 16):
                    rv[r, pl.ds(k2 * 16, 16)] = rv[r, pl.ds(k2 * 16, 16)] * g

        for v in range(CH // 16):
            sidx_small[pl.ds(v * 16, 16)] = sidx_all[pl.ds(cc * CH + v * 16, 16)]
        pltpu.sync_copy(rv, acc_sh.at[sidx_small], add=True)

    start_gather(0, 0)

    @pl.loop(0, NFULL, step=2)
    def _main(ci):
        wait_gather(0)
        start_gather(ci + 1, 1)
        process(ci, 0)
        wait_gather(1)

        @pl.when(ci + 2 < NFULL)
        def _():
            start_gather(ci + 2, 0)

        process(ci + 1, 1)

    pltpu.async_copy(table_hbm.at[gidx_all.at[pl.ds(NFULL * CH, REM)]],
                     rows0.at[pl.ds(0, REM)], semg0).wait()
    g16 = gate_all[pl.ds(NFULL * CH, 16)]
    for j in range(16):
        g = g16[j]
        for k2 in range(D // 16):
            rows0[j, pl.ds(k2 * 16, 16)] = rows0[j, pl.ds(k2 * 16, 16)] * g
    sidx_rem[pl.ds(0, 16)] = sidx_all[pl.ds(NFULL * CH, 16)]
    pltpu.sync_copy(rows0.at[pl.ds(0, REM)], acc_sh.at[sidx_rem], add=True)

    plsc.subcore_barrier()

    @pl.loop(0, (n_row_chunks + NS - 1) // NS)
    def _dump(k):
        rc = s + k * NS

        @pl.when(rc < n_row_chunks)
        def _():
            pltpu.sync_copy(acc_sh.at[pl.ds(rc * ZCH, ZCH)],
                            out_hbm.at[c, pl.ds(rc * ZCH, ZCH)])


@functools.lru_cache(maxsize=1)
def _make_seg_sum_sc():
    return pl.kernel(
        _seg_sum_body,
        out_type=jax.ShapeDtypeStruct((NC, N_NODES, D), jnp.float32),
        mesh=plsc.VectorSubcoreMesh(core_axis_name="c", subcore_axis_name="s"),
        scratch_types=[
            pltpu.VMEM((EPW,), jnp.int32),
            pltpu.VMEM((EPW,), jnp.int32),
            pltpu.VMEM((EPW,), jnp.float32),
            pltpu.VMEM((CH, D), jnp.float32),
            pltpu.VMEM((CH, D), jnp.float32),
            pltpu.VMEM((CH,), jnp.int32),
            pltpu.VMEM((REM,), jnp.int32),
            pltpu.VMEM_SHARED((N_NODES, D), jnp.float32),
            pltpu.SemaphoreType.DMA,
            pltpu.SemaphoreType.DMA,
        ],
    )


def _segment_sum_sc(table, gather_idx, scatter_idx, gate):
    return _make_seg_sum_sc()(table, gather_idx, scatter_idx, gate)


ROWB = 400


def _embed_body(x_ref, w_ref, b_ref, out_ref):
    out_ref[...] = jnp.dot(x_ref[...], w_ref[...],
                           preferred_element_type=jnp.float32) + b_ref[...]


def _embed(x, W_lin, b_lin):
    grid = (N_NODES // ROWB,)
    return pl.pallas_call(
        _embed_body,
        grid=grid,
        in_specs=[
            pl.BlockSpec((ROWB, D), lambda i: (i, 0)),
            pl.BlockSpec((D, D), lambda i: (0, 0)),
            pl.BlockSpec((1, D), lambda i: (0, 0)),
        ],
        out_specs=pl.BlockSpec((ROWB, D), lambda i: (i, 0)),
        out_shape=jax.ShapeDtypeStruct((N_NODES, D), jnp.float32),
    )(x, W_lin, b_lin.reshape(1, D))


EB = 32000


def _gate_body(c0, c1, c2, c3, w_ref, out_ref):
    z = (c0[...] * w_ref[0, 0] + c1[...] * w_ref[0, 1]
         + c2[...] * w_ref[0, 2] + c3[...] * w_ref[0, 3] + w_ref[0, 4])
    out_ref[...] = 1.0 / (1.0 + jnp.exp(-z))


def _gate(edge_attr, w_e, b_e):
    rows = N_EDGES // 128
    cols = [edge_attr[:, k].reshape(rows, 128) for k in range(4)]
    wb = jnp.concatenate([w_e.reshape(-1), b_e.reshape(-1)]).reshape(1, 5)
    g2 = pl.pallas_call(
        _gate_body,
        in_specs=[pl.BlockSpec((rows, 128), lambda: (0, 0))] * 4
        + [pl.BlockSpec(memory_space=pltpu.MemorySpace.SMEM)],
        out_specs=pl.BlockSpec((rows, 128), lambda: (0, 0)),
        out_shape=jax.ShapeDtypeStruct((rows, 128), jnp.float32),
    )(*cols, wb)
    return g2.reshape(N_EDGES)


def _mid_body(x0_ref, m_ref, w1a_ref, w1b_ref, b1_ref, wc1_ref,
              g0_ref, h1_ref, s1_ref, x1_ref, l0_ref):
    i = pl.program_id(0)
    msg = m_ref[0] + m_ref[1]
    h1 = jnp.dot(x0_ref[...], w1a_ref[...], preferred_element_type=jnp.float32)
    h1 += jnp.dot(msg, w1b_ref[...], preferred_element_type=jnp.float32)
    h1 = jnp.maximum(h1 + b1_ref[...], 0.0)
    h1_ref[...] = h1
    logits = jnp.dot(h1, wc1_ref[...], preferred_element_type=jnp.float32)
    col = lax.broadcasted_iota(jnp.int32, logits.shape, 1)
    logits = jnp.where(col < K0, logits, -jnp.inf)
    m = jnp.max(logits, axis=-1, keepdims=True)
    e = jnp.exp(logits - m)
    s1 = e / jnp.sum(e, axis=-1, keepdims=True)
    s1_ref[...] = s1
    d = s1 - g0_ref[...]
    part = jnp.sum(d * d)
    x1_blk = jnp.dot(s1.T, h1, preferred_element_type=jnp.float32)

    @pl.when(i == 0)
    def _():
        x1_ref[...] = x1_blk
        l0_ref[0, 0] = part

    @pl.when(i != 0)
    def _():
        x1_ref[...] += x1_blk
        l0_ref[0, 0] += part


def _mid(x0, msg2, W1, b1, Wc1p, g0p):
    grid = (N_NODES // ROWB,)
    return pl.pallas_call(
        _mid_body,
        grid=grid,
        in_specs=[
            pl.BlockSpec((ROWB, D), lambda i: (i, 0)),
            pl.BlockSpec((NC, ROWB, D), lambda i: (0, i, 0)),
            pl.BlockSpec((D, D), lambda i: (0, 0)),
            pl.BlockSpec((D, D), lambda i: (0, 0)),
            pl.BlockSpec((1, D), lambda i: (0, 0)),
            pl.BlockSpec((D, D), lambda i: (0, 0)),
            pl.BlockSpec((ROWB, D), lambda i: (i, 0)),
        ],
        out_specs=[
            pl.BlockSpec((ROWB, D), lambda i: (i, 0)),
            pl.BlockSpec((ROWB, D), lambda i: (i, 0)),
            pl.BlockSpec((D, D), lambda i: (0, 0)),
            pl.BlockSpec(memory_space=pltpu.MemorySpace.SMEM),
        ],
        out_shape=[
            jax.ShapeDtypeStruct((N_NODES, D), jnp.float32),
            jax.ShapeDtypeStruct((N_NODES, D), jnp.float32),
            jax.ShapeDtypeStruct((D, D), jnp.float32),
            jax.ShapeDtypeStruct((1, 1), jnp.float32),
        ],
    )(x0, msg2, W1[:D], W1[D:], b1.reshape(1, D), Wc1p, g0p)


def _fin_body(s1_ref, b_ref, x1_ref, w2a_ref, w2b_ref, b2_ref,
              wc2_ref, g1_ref, a1_ref, x2_ref, l1_ref, acc):
    i = pl.program_id(0)
    nblk = pl.num_programs(0)
    bsum = b_ref[0] + b_ref[1]
    blk = jnp.dot(s1_ref[...].T, bsum, preferred_element_type=jnp.float32)

    @pl.when(i == 0)
    def _():
        acc[...] = blk

    @pl.when(i != 0)
    def _():
        acc[...] += blk

    @pl.when(i == nblk - 1)
    def _():
        a1 = acc[...]
        a1_ref[...] = a1
        x1 = x1_ref[...]
        msg2 = jnp.dot(a1, x1, preferred_element_type=jnp.float32)
        h2 = jnp.dot(x1, w2a_ref[...], preferred_element_type=jnp.float32)
        h2 += jnp.dot(msg2, w2b_ref[...], preferred_element_type=jnp.float32)
        h2 = jnp.maximum(h2 + b2_ref[...], 0.0)
        row = lax.broadcasted_iota(jnp.int32, h2.shape, 0)
        h2 = jnp.where(row < K0, h2, 0.0)
        logits = jnp.dot(h2, wc2_ref[...], preferred_element_type=jnp.float32)
        col = lax.broadcasted_iota(jnp.int32, logits.shape, 1)
        valid = (col < K1) & (row < K0)
        logits = jnp.where(col < K1, logits, -jnp.inf)
        m = jnp.max(logits, axis=-1, keepdims=True)
        e = jnp.exp(logits - m)
        s2 = e / jnp.sum(e, axis=-1, keepdims=True)
        s2 = jnp.where(row < K0, s2, 0.0)
        d = jnp.where(valid, s2 - g1_ref[...], 0.0)
        l1_ref[0, 0] = jnp.sum(d * d) / (K0 * K1)
        x2_ref[...] = jnp.dot(s2.T, h2, preferred_element_type=jnp.float32)


def _final(s1p, bparts, x1, W2, b2, Wc2p, g1p):
    grid = (N_NODES // ROWB,)
    return pl.pallas_call(
        _fin_body,
        grid=grid,
        in_specs=[
            pl.BlockSpec((ROWB, D), lambda i: (i, 0)),
            pl.BlockSpec((NC, ROWB, D), lambda i: (0, i, 0)),
            pl.BlockSpec((D, D), lambda i: (0, 0)),
            pl.BlockSpec((D, D), lambda i: (0, 0)),
            pl.BlockSpec((D, D), lambda i: (0, 0)),
            pl.BlockSpec((1, D), lambda i: (0, 0)),
            pl.BlockSpec((D, D), lambda i: (0, 0)),
            pl.BlockSpec((D, D), lambda i: (0, 0)),
        ],
        out_specs=[
            pl.BlockSpec((D, D), lambda i: (0, 0)),
            pl.BlockSpec((D, D), lambda i: (0, 0)),
            pl.BlockSpec(memory_space=pltpu.MemorySpace.SMEM),
        ],
        out_shape=[
            jax.ShapeDtypeStruct((D, D), jnp.float32),
            jax.ShapeDtypeStruct((D, D), jnp.float32),
            jax.ShapeDtypeStruct((1, 1), jnp.float32),
        ],
        scratch_shapes=[pltpu.VMEM((D, D), jnp.float32)],
    )(s1p, bparts, x1, W2[:D], W2[D:], b2.reshape(1, D), Wc2p, g1p)


def kernel(x, edge_index, edge_attr, g_true_0, g_true_1, W_lin, b_lin,
           w_e, b_e, W1, b1, Wc1, W2, b2, Wc2):
    src = edge_index[0].astype(jnp.int32)
    dst = edge_index[1].astype(jnp.int32)

    x0 = _embed(x, W_lin, b_lin)
    gate = _gate(edge_attr, w_e, b_e)

    msg2 = _segment_sum_sc(x0, src, dst, gate)

    Wc1p = jnp.pad(Wc1, ((0, 0), (0, D - K0)))
    g0p = jnp.pad(g_true_0, ((0, 0), (0, D - K0)))
    h1, s1p, x1, l0 = _mid(x0, msg2, W1, b1, Wc1p, g0p)
    loss0 = l0[0, 0] / (N_NODES * K0)

    bparts = _segment_sum_sc(s1p, dst, src, gate)

    Wc2p = jnp.pad(Wc2, ((0, 0), (0, D - K1)))
    g1p = jnp.pad(g_true_1, ((0, D - K0), (0, D - K1)))
    a1, x2p, l1 = _final(s1p, bparts, x1, W2, b2, Wc2p, g1p)

    x2 = x2p[:K1, :]
    return jnp.concatenate([x2.reshape(-1), jnp.stack([loss0, l1[0, 0]])])


# speedup vs baseline: 7.9329x; 1.1019x over previous
import functools

import jax
import jax.numpy as jnp
from jax import lax
from jax.experimental import pallas as pl
from jax.experimental.pallas import tpu as pltpu
from jax.experimental.pallas import tpu_sc as plsc

N_NODES = 10000
N_EDGES = 320000
D = 128
K0 = 100
K1 = 10

NC = 2
NS = 16
NW = NC * NS
EPW = N_EDGES // NW
CH = 64
NFULL = EPW // CH
REM = EPW - NFULL * CH
ZCH = 80


def _seg_sum_body(table_hbm, gidx_hbm, sidx_hbm, gate_hbm, out_hbm,
                  gidx_all, sidx_all, gate_all, rows0, rows1, sidx_b0,
                  sidx_b1, sidx_rem, acc_sh, semg0, semg1, sems0, sems1):
    c = lax.axis_index("c")
    s = lax.axis_index("s")
    wid = c * NS + s
    ebase = wid * EPW

    pltpu.sync_copy(gidx_hbm.at[pl.ds(ebase, EPW)], gidx_all)
    pltpu.sync_copy(sidx_hbm.at[pl.ds(ebase, EPW)], sidx_all)
    pltpu.sync_copy(gate_hbm.at[pl.ds(ebase, EPW)], gate_all)

    zero16 = jnp.zeros((16,), jnp.float32)

    @pl.loop(0, ZCH)
    def _zero_rows(r):
        for j in range(D // 16):
            rows0[r, pl.ds(j * 16, 16)] = zero16

    n_row_chunks = N_NODES // ZCH

    @pl.loop(0, (n_row_chunks + NS - 1) // NS)
    def _zero_acc(k):
        rc = s + k * NS

        @pl.when(rc < n_row_chunks)
        def _():
            pltpu.sync_copy(rows0.at[pl.ds(0, ZCH)], acc_sh.at[pl.ds(rc * ZCH, ZCH)])

    plsc.subcore_barrier()

    rows = (rows0, rows1)
    semg = (semg0, semg1)
    sems = (sems0, sems1)
    sidx = (sidx_b0, sidx_b1)

    def start_gather(cc, b):
        pltpu.async_copy(table_hbm.at[gidx_all.at[pl.ds(cc * CH, CH)]],
                         rows[b], semg[b])

    def wait_gather(b):
        pltpu.make_async_copy(table_hbm.at[gidx_all.at[pl.ds(0, CH)]],
                              rows[b], semg[b]).wait()

    def wait_scatter(b):
        pltpu.make_async_copy(rows[b], acc_sh.at[sidx[b]], sems[b]).wait()

    def scale_and_scatter(cc, b):
        rv = rows[b]

        @pl.loop(0, CH // 16)
        def _scale(rb):
            g16 = gate_all[pl.ds(cc * CH + rb * 16, 16)]
            for j in range(16):
                g = g16[j]
                r = rb * 16 + j
                for k2 in range(D // 16):
                    rv[r, pl.ds(k2 * 16, 16)] = rv[r, pl.ds(k2 * 16, 16)] * g

        sb = sidx[b]
        for v in range(CH // 16):
            sb[pl.ds(v * 16, 16)] = sidx_all[pl.ds(cc * CH + v * 16, 16)]
        pltpu.async_copy(rv, acc_sh.at[sb], sems[b], add=True)

    start_gather(0, 0)
    start_gather(1, 1)
    wait_gather(0)
    scale_and_scatter(0, 0)

    def step(cc, b):
        wait_scatter(1 - b)
        start_gather(cc + 1, 1 - b)
        wait_gather(b)
        scale_and_scatter(cc, b)

    @pl.loop(1, NFULL - 1, step=2)
    def _main(ci):
        step(ci, 1)
        step(ci + 1, 0)

    wait_scatter(0)
    wait_gather(1)
    scale_and_scatter(NFULL - 1, 1)

    pltpu.async_copy(table_hbm.at[gidx_all.at[pl.ds(NFULL * CH, REM)]],
                     rows0.at[pl.ds(0, REM)], semg0).wait()
    g16 = gate_all[pl.ds(NFULL * CH, 16)]
    for j in range(16):
        g = g16[j]
        for k2 in range(D // 16):
            rows0[j, pl.ds(k2 * 16, 16)] = rows0[j, pl.ds(k2 * 16, 16)] * g
    sidx_rem[pl.ds(0, 16)] = sidx_all[pl.ds(NFULL * CH, 16)]
    pltpu.sync_copy(rows0.at[pl.ds(0, REM)], acc_sh.at[sidx_rem], add=True)
    wait_scatter(1)

    plsc.subcore_barrier()

    @pl.loop(0, (n_row_chunks + NS - 1) // NS)
    def _dump(k):
        rc = s + k * NS

        @pl.when(rc < n_row_chunks)
        def _():
            pltpu.sync_copy(acc_sh.at[pl.ds(rc * ZCH, ZCH)],
                            out_hbm.at[c, pl.ds(rc * ZCH, ZCH)])


@functools.lru_cache(maxsize=1)
def _make_seg_sum_sc():
    return pl.kernel(
        _seg_sum_body,
        out_type=jax.ShapeDtypeStruct((NC, N_NODES, D), jnp.float32),
        mesh=plsc.VectorSubcoreMesh(core_axis_name="c", subcore_axis_name="s"),
        scratch_types=[
            pltpu.VMEM((EPW,), jnp.int32),
            pltpu.VMEM((EPW,), jnp.int32),
            pltpu.VMEM((EPW,), jnp.float32),
            pltpu.VMEM((CH, D), jnp.float32),
            pltpu.VMEM((CH, D), jnp.float32),
            pltpu.VMEM((CH,), jnp.int32),
            pltpu.VMEM((CH,), jnp.int32),
            pltpu.VMEM((REM,), jnp.int32),
            pltpu.VMEM_SHARED((N_NODES, D), jnp.float32),
            pltpu.SemaphoreType.DMA,
            pltpu.SemaphoreType.DMA,
            pltpu.SemaphoreType.DMA,
            pltpu.SemaphoreType.DMA,
        ],
    )


def _segment_sum_sc(table, gather_idx, scatter_idx, gate):
    return _make_seg_sum_sc()(table, gather_idx, scatter_idx, gate)


ROWB = 400


def _embed_body(x_ref, w_ref, b_ref, out_ref):
    out_ref[...] = jnp.dot(x_ref[...], w_ref[...],
                           preferred_element_type=jnp.float32) + b_ref[...]


def _embed(x, W_lin, b_lin):
    grid = (N_NODES // ROWB,)
    return pl.pallas_call(
        _embed_body,
        grid=grid,
        in_specs=[
            pl.BlockSpec((ROWB, D), lambda i: (i, 0)),
            pl.BlockSpec((D, D), lambda i: (0, 0)),
            pl.BlockSpec((1, D), lambda i: (0, 0)),
        ],
        out_specs=pl.BlockSpec((ROWB, D), lambda i: (i, 0)),
        out_shape=jax.ShapeDtypeStruct((N_NODES, D), jnp.float32),
    )(x, W_lin, b_lin.reshape(1, D))


EB = 32000


def _gate_body(c0, c1, c2, c3, w_ref, out_ref):
    z = (c0[...] * w_ref[0, 0] + c1[...] * w_ref[0, 1]
         + c2[...] * w_ref[0, 2] + c3[...] * w_ref[0, 3] + w_ref[0, 4])
    out_ref[...] = 1.0 / (1.0 + jnp.exp(-z))


def _gate(edge_attr, w_e, b_e):
    rows = N_EDGES // 128
    cols = [edge_attr[:, k].reshape(rows, 128) for k in range(4)]
    wb = jnp.concatenate([w_e.reshape(-1), b_e.reshape(-1)]).reshape(1, 5)
    g2 = pl.pallas_call(
        _gate_body,
        in_specs=[pl.BlockSpec((rows, 128), lambda: (0, 0))] * 4
        + [pl.BlockSpec(memory_space=pltpu.MemorySpace.SMEM)],
        out_specs=pl.BlockSpec((rows, 128), lambda: (0, 0)),
        out_shape=jax.ShapeDtypeStruct((rows, 128), jnp.float32),
    )(*cols, wb)
    return g2.reshape(N_EDGES)


def _mid_body(x0_ref, m_ref, w1a_ref, w1b_ref, b1_ref, wc1_ref,
              g0_ref, h1_ref, s1_ref, x1_ref, l0_ref):
    i = pl.program_id(0)
    msg = m_ref[0] + m_ref[1]
    h1 = jnp.dot(x0_ref[...], w1a_ref[...], preferred_element_type=jnp.float32)
    h1 += jnp.dot(msg, w1b_ref[...], preferred_element_type=jnp.float32)
    h1 = jnp.maximum(h1 + b1_ref[...], 0.0)
    h1_ref[...] = h1
    logits = jnp.dot(h1, wc1_ref[...], preferred_element_type=jnp.float32)
    col = lax.broadcasted_iota(jnp.int32, logits.shape, 1)
    logits = jnp.where(col < K0, logits, -jnp.inf)
    m = jnp.max(logits, axis=-1, keepdims=True)
    e = jnp.exp(logits - m)
    s1 = e / jnp.sum(e, axis=-1, keepdims=True)
    s1_ref[...] = s1
    d = s1 - g0_ref[...]
    part = jnp.sum(d * d)
    x1_blk = jnp.dot(s1.T, h1, preferred_element_type=jnp.float32)

    @pl.when(i == 0)
    def _():
        x1_ref[...] = x1_blk
        l0_ref[0, 0] = part

    @pl.when(i != 0)
    def _():
        x1_ref[...] += x1_blk
        l0_ref[0, 0] += part


def _mid(x0, msg2, W1, b1, Wc1p, g0p):
    grid = (N_NODES // ROWB,)
    return pl.pallas_call(
        _mid_body,
        grid=grid,
        in_specs=[
            pl.BlockSpec((ROWB, D), lambda i: (i, 0)),
            pl.BlockSpec((NC, ROWB, D), lambda i: (0, i, 0)),
            pl.BlockSpec((D, D), lambda i: (0, 0)),
            pl.BlockSpec((D, D), lambda i: (0, 0)),
            pl.BlockSpec((1, D), lambda i: (0, 0)),
            pl.BlockSpec((D, D), lambda i: (0, 0)),
            pl.BlockSpec((ROWB, D), lambda i: (i, 0)),
        ],
        out_specs=[
            pl.BlockSpec((ROWB, D), lambda i: (i, 0)),
            pl.BlockSpec((ROWB, D), lambda i: (i, 0)),
            pl.BlockSpec((D, D), lambda i: (0, 0)),
            pl.BlockSpec(memory_space=pltpu.MemorySpace.SMEM),
        ],
        out_shape=[
            jax.ShapeDtypeStruct((N_NODES, D), jnp.float32),
            jax.ShapeDtypeStruct((N_NODES, D), jnp.float32),
            jax.ShapeDtypeStruct((D, D), jnp.float32),
            jax.ShapeDtypeStruct((1, 1), jnp.float32),
        ],
    )(x0, msg2, W1[:D], W1[D:], b1.reshape(1, D), Wc1p, g0p)


def _fin_body(s1_ref, b_ref, x1_ref, w2a_ref, w2b_ref, b2_ref,
              wc2_ref, g1_ref, a1_ref, x2_ref, l1_ref, acc):
    i = pl.program_id(0)
    nblk = pl.num_programs(0)
    bsum = b_ref[0] + b_ref[1]
    blk = jnp.dot(s1_ref[...].T, bsum, preferred_element_type=jnp.float32)

    @pl.when(i == 0)
    def _():
        acc[...] = blk

    @pl.when(i != 0)
    def _():
        acc[...] += blk

    @pl.when(i == nblk - 1)
    def _():
        a1 = acc[...]
        a1_ref[...] = a1
        x1 = x1_ref[...]
        msg2 = jnp.dot(a1, x1, preferred_element_type=jnp.float32)
        h2 = jnp.dot(x1, w2a_ref[...], preferred_element_type=jnp.float32)
        h2 += jnp.dot(msg2, w2b_ref[...], preferred_element_type=jnp.float32)
        h2 = jnp.maximum(h2 + b2_ref[...], 0.0)
        row = lax.broadcasted_iota(jnp.int32, h2.shape, 0)
        h2 = jnp.where(row < K0, h2, 0.0)
        logits = jnp.dot(h2, wc2_ref[...], preferred_element_type=jnp.float32)
        col = lax.broadcasted_iota(jnp.int32, logits.shape, 1)
        valid = (col < K1) & (row < K0)
        logits = jnp.where(col < K1, logits, -jnp.inf)
        m = jnp.max(logits, axis=-1, keepdims=True)
        e = jnp.exp(logits - m)
        s2 = e / jnp.sum(e, axis=-1, keepdims=True)
        s2 = jnp.where(row < K0, s2, 0.0)
        d = jnp.where(valid, s2 - g1_ref[...], 0.0)
        l1_ref[0, 0] = jnp.sum(d * d) / (K0 * K1)
        x2_ref[...] = jnp.dot(s2.T, h2, preferred_element_type=jnp.float32)


def _final(s1p, bparts, x1, W2, b2, Wc2p, g1p):
    grid = (N_NODES // ROWB,)
    return pl.pallas_call(
        _fin_body,
        grid=grid,
        in_specs=[
            pl.BlockSpec((ROWB, D), lambda i: (i, 0)),
            pl.BlockSpec((NC, ROWB, D), lambda i: (0, i, 0)),
            pl.BlockSpec((D, D), lambda i: (0, 0)),
            pl.BlockSpec((D, D), lambda i: (0, 0)),
            pl.BlockSpec((D, D), lambda i: (0, 0)),
            pl.BlockSpec((1, D), lambda i: (0, 0)),
            pl.BlockSpec((D, D), lambda i: (0, 0)),
            pl.BlockSpec((D, D), lambda i: (0, 0)),
        ],
        out_specs=[
            pl.BlockSpec((D, D), lambda i: (0, 0)),
            pl.BlockSpec((D, D), lambda i: (0, 0)),
            pl.BlockSpec(memory_space=pltpu.MemorySpace.SMEM),
        ],
        out_shape=[
            jax.ShapeDtypeStruct((D, D), jnp.float32),
            jax.ShapeDtypeStruct((D, D), jnp.float32),
            jax.ShapeDtypeStruct((1, 1), jnp.float32),
        ],
        scratch_shapes=[pltpu.VMEM((D, D), jnp.float32)],
    )(s1p, bparts, x1, W2[:D], W2[D:], b2.reshape(1, D), Wc2p, g1p)


def kernel(x, edge_index, edge_attr, g_true_0, g_true_1, W_lin, b_lin,
           w_e, b_e, W1, b1, Wc1, W2, b2, Wc2):
    src = edge_index[0].astype(jnp.int32)
    dst = edge_index[1].astype(jnp.int32)

    x0 = _embed(x, W_lin, b_lin)
    gate = _gate(edge_attr, w_e, b_e)

    msg2 = _segment_sum_sc(x0, src, dst, gate)

    Wc1p = jnp.pad(Wc1, ((0, 0), (0, D - K0)))
    g0p = jnp.pad(g_true_0, ((0, 0), (0, D - K0)))
    h1, s1p, x1, l0 = _mid(x0, msg2, W1, b1, Wc1p, g0p)
    loss0 = l0[0, 0] / (N_NODES * K0)

    bparts = _segment_sum_sc(s1p, dst, src, gate)

    Wc2p = jnp.pad(Wc2, ((0, 0), (0, D - K1)))
    g1p = jnp.pad(g_true_1, ((0, D - K0), (0, D - K1)))
    a1, x2p, l1 = _final(s1p, bparts, x1, W2, b2, Wc2p, g1p)

    x2 = x2p[:K1, :]
    return jnp.concatenate([x2.reshape(-1), jnp.stack([loss0, l1[0, 0]])])
